# baseline (speedup 1.0000x reference)
import functools

import jax
import jax.numpy as jnp
from jax import lax
from jax.experimental import pallas as pl
from jax.experimental.pallas import tpu as pltpu
from jax.experimental.pallas import tpu_sc as plsc

N = 10000
E = 320000
D = 128
HD = D // 2
G = 64

NC = 2
NS = 16
NW = NC * NS

CHR = 100
ROWS = E // CHR
RPT = N // NS

DEG_CPT = ROWS // NW
DEG_NB = 5
DEG_NGRP = DEG_CPT // DEG_NB

EDG_CPT = ROWS // NS
EDG_NPH = 4
EDG_PCH = EDG_CPT // EDG_NPH
EDG_NB = 10
EDG_NGRP = EDG_PCH // EDG_NB
CO_K = 5
CO_R = RPT // CO_K

BN = 1000
NBLK = N // BN

_mesh = plsc.VectorSubcoreMesh(core_axis_name="c", subcore_axis_name="s")
_sc_params = pltpu.CompilerParams(use_tc_tiling_on_sc=False)


@functools.partial(
    pl.kernel,
    out_type=jax.ShapeDtypeStruct((NC, N, 16), jnp.float32),
    mesh=_mesh,
    scratch_types=[
        pltpu.VMEM((DEG_CPT, CHR), jnp.int32),
        pltpu.VMEM((CHR, 16), jnp.float32),
        pltpu.VMEM_SHARED((N, 16), jnp.float32),
        pltpu.SemaphoreType.DMA,
        pltpu.SemaphoreType.DMA,
        pltpu.SemaphoreType.DMA,
        pltpu.SemaphoreType.DMA,
        pltpu.SemaphoreType.DMA,
    ],
    compiler_params=_sc_params,
)
def _sc_degree(dst_hbm, ones_hbm, zeros_hbm, out_hbm,
               idxd_v, ones_v, acc_sh, s0, s1, s2, s3, s4):
    ssem = (s0, s1, s2, s3, s4)
    cid = lax.axis_index("c")
    sid = lax.axis_index("s")
    wid = sid * NC + cid
    rbase = sid * RPT
    pltpu.sync_copy(zeros_hbm.at[pl.ds(rbase, RPT)],
                    acc_sh.at[pl.ds(rbase, RPT)])
    pltpu.sync_copy(ones_hbm, ones_v)
    pltpu.sync_copy(dst_hbm.at[pl.ds(wid * DEG_CPT, DEG_CPT)], idxd_v)
    plsc.subcore_barrier()

    def scat(c, b):
        pltpu.async_copy(ones_v, acc_sh.at[idxd_v.at[c]], ssem[b], add=True)

    def swait(c, b):
        pltpu.make_async_copy(ones_v, acc_sh.at[idxd_v.at[c]], ssem[b]).wait()

    def group(j0, carry):
        j = j0 * DEG_NB
        for b in range(DEG_NB):
            scat(j + b, b)
        for b in range(DEG_NB):
            swait(j + b, b)
        return carry

    lax.fori_loop(0, DEG_NGRP, group, 0)

    plsc.subcore_barrier()
    pltpu.sync_copy(acc_sh.at[pl.ds(rbase, RPT)],
                    out_hbm.at[cid, pl.ds(rbase, RPT)])


@functools.partial(
    pl.kernel,
    out_type=jax.ShapeDtypeStruct((2 * N, HD), jnp.float32),
    mesh=_mesh,
    scratch_types=[
        pltpu.VMEM((EDG_PCH, CHR), jnp.int32),
        pltpu.VMEM((EDG_PCH, CHR), jnp.int32),
        pltpu.VMEM((EDG_NB, CHR, HD), jnp.float32),
        pltpu.VMEM((CO_K, CO_R), jnp.int32),
        pltpu.VMEM((CO_R, HD), jnp.float32),
        pltpu.VMEM_SHARED((N, HD), jnp.float32),
    ] + [pltpu.SemaphoreType.DMA] * (2 * EDG_NB),
    compiler_params=_sc_params,
)
def _sc_edge_sum(y2n_hbm, src2_hbm, dst_hbm, coidx_hbm, zeros_hbm, out_hbm,
                 idxs_v, idxd_v, rows_v, coidx_v, stage_v, acc_sh,
                 *sems):
    gsem = sems[:EDG_NB]
    ssem = sems[EDG_NB:]
    cid = lax.axis_index("c")
    sid = lax.axis_index("s")
    rbase = sid * RPT
    pltpu.sync_copy(zeros_hbm.at[pl.ds(rbase, RPT)],
                    acc_sh.at[pl.ds(rbase, RPT)])
    pltpu.sync_copy(coidx_hbm.at[cid, sid], coidx_v)
    cbase = sid * EDG_CPT

    def gath(c, b):
        pltpu.async_copy(y2n_hbm.at[idxs_v.at[c]], rows_v.at[b], gsem[b])

    def gwait(c, b):
        pltpu.make_async_copy(y2n_hbm.at[idxs_v.at[c]], rows_v.at[b],
                              gsem[b]).wait()

    def scat(c, b):
        pltpu.async_copy(rows_v.at[b], acc_sh.at[idxd_v.at[c]], ssem[b],
                         add=True)

    def swait(c, b):
        pltpu.make_async_copy(rows_v.at[b], acc_sh.at[idxd_v.at[c]],
                              ssem[b]).wait()

    for p in range(EDG_NPH):
        pbase = cbase + p * EDG_PCH
        pltpu.sync_copy(src2_hbm.at[cid, pl.ds(pbase, EDG_PCH)], idxs_v)
        pltpu.sync_copy(dst_hbm.at[pl.ds(pbase, EDG_PCH)], idxd_v)
        if p == 0:
            plsc.subcore_barrier()

        for b in range(EDG_NB):
            gath(b, b)

        def group(j0, carry):
            j = j0 * EDG_NB
            for b in range(EDG_NB):
                gwait(j + b, b)
                scat(j + b, b)
            for b in range(EDG_NB):
                swait(j + b, b)
                gath(j + EDG_NB + b, b)
            return carry

        lax.fori_loop(0, EDG_NGRP - 1, group, 0)
        j = (EDG_NGRP - 1) * EDG_NB
        for b in range(EDG_NB):
            gwait(j + b, b)
            scat(j + b, b)
        for b in range(EDG_NB):
            swait(j + b, b)

    plsc.subcore_barrier()
    for k in range(CO_K):
        pltpu.sync_copy(acc_sh.at[pl.ds(rbase + k * CO_R, CO_R)], stage_v)
        pltpu.async_copy(stage_v, out_hbm.at[coidx_v.at[k]], ssem[0]).wait()


def _dvec(degp_ref):
    deg = 1.0 + degp_ref[0, :, :1] + degp_ref[1, :, :1]
    return lax.rsqrt(deg)


def _tc1_body(degp_ref, x_ref, w1_ref, y1_ref):
    d = _dvec(degp_ref)
    xw = jnp.dot(x_ref[...], w1_ref[...],
                 preferred_element_type=jnp.float32,
                 precision=lax.Precision.HIGHEST)
    y1_ref[...] = xw * d


def _tc2_body(degp_ref, s_ref, y1_ref, w2_ref, b1_ref, y2_ref):
    d = _dvec(degp_ref)
    s = s_ref[...] + y1_ref[...]
    h = jnp.maximum(d * s + b1_ref[...], 0.0)
    hw = jnp.dot(h, w2_ref[...],
                 preferred_element_type=jnp.float32,
                 precision=lax.Precision.HIGHEST)
    y2_ref[...] = hw * d


def _tc3_body(degp_ref, s_ref, y2_ref, b2_ref, batch_ref, out_ref, acc, cnt):
    i = pl.program_id(0)

    @pl.when(i == 0)
    def _init():
        acc[...] = jnp.zeros_like(acc)
        cnt[...] = jnp.zeros_like(cnt)

    d = _dvec(degp_ref)
    o = d * (s_ref[...] + y2_ref[...]) + b2_ref[...]
    seg = batch_ref[0, 0, :]
    oh = (lax.broadcasted_iota(jnp.int32, (G, BN), 0)
          == seg[None, :]).astype(jnp.float32)
    acc[...] += jnp.dot(oh, o, preferred_element_type=jnp.float32,
                        precision=lax.Precision.HIGHEST)
    cnt[...] += jnp.sum(oh, axis=1, keepdims=True)

    @pl.when(i == NBLK - 1)
    def _fin():
        out_ref[...] = acc[...] / jnp.maximum(cnt[...], 1.0)


_row = lambda i: (i, 0)
_fix2 = lambda i: (0, 0)
_split3 = lambda i: (0, i, 0)

_tc1 = pl.pallas_call(
    _tc1_body,
    grid=(NBLK,),
    in_specs=[
        pl.BlockSpec((NC, BN, 16), _split3),
        pl.BlockSpec((BN, D), _row),
        pl.BlockSpec((D, D), _fix2),
    ],
    out_specs=pl.BlockSpec((BN, D), _row),
    out_shape=jax.ShapeDtypeStruct((N, D), jnp.float32),
)

_tc2 = pl.pallas_call(
    _tc2_body,
    grid=(NBLK,),
    in_specs=[
        pl.BlockSpec((NC, BN, 16), _split3),
        pl.BlockSpec((BN, D), _row),
        pl.BlockSpec((BN, D), _row),
        pl.BlockSpec((D, D), _fix2),
        pl.BlockSpec((1, D), _fix2),
    ],
    out_specs=pl.BlockSpec((BN, D), _row),
    out_shape=jax.ShapeDtypeStruct((N, D), jnp.float32),
)

_tc3 = pl.pallas_call(
    _tc3_body,
    grid=(NBLK,),
    in_specs=[
        pl.BlockSpec((NC, BN, 16), _split3),
        pl.BlockSpec((BN, D), _row),
        pl.BlockSpec((BN, D), _row),
        pl.BlockSpec((1, D), _fix2),
        pl.BlockSpec((1, 1, BN), lambda i: (i, 0, 0)),
    ],
    out_specs=pl.BlockSpec((G, D), _fix2),
    out_shape=jax.ShapeDtypeStruct((G, D), jnp.float32),
    scratch_shapes=[
        pltpu.VMEM((G, D), jnp.float32),
        pltpu.VMEM((G, 1), jnp.float32),
    ],
)


def kernel(x, edge_index, batch, W1, b1, W2, b2):
    x = x.astype(jnp.float32)
    src2d = edge_index[0].reshape(ROWS, CHR)
    dst2d = edge_index[1].reshape(ROWS, CHR)
    src2 = jnp.stack([src2d * 2, src2d * 2 + 1])
    co = 2 * jnp.arange(N, dtype=jnp.int32).reshape(NS, CO_K, CO_R)
    coidx = jnp.stack([co, co + 1])
    ones16 = jnp.ones((CHR, 16), jnp.float32)
    zeros16 = jnp.zeros((N, 16), jnp.float32)
    zerosH = jnp.zeros((N, HD), jnp.float32)
    b1r = b1.reshape(1, D)
    b2r = b2.reshape(1, D)
    batch3 = batch.reshape(NBLK, 1, BN)

    degp = _sc_degree(dst2d, ones16, zeros16)
    y1 = _tc1(degp, x, W1)
    s1 = _sc_edge_sum(y1.reshape(2 * N, HD), src2, dst2d, coidx,
                      zerosH).reshape(N, D)
    y2 = _tc2(degp, s1, y1, W2, b1r)
    s2 = _sc_edge_sum(y2.reshape(2 * N, HD), src2, dst2d, coidx,
                      zerosH).reshape(N, D)
    out = _tc3(degp, s2, y2, b2r, batch3)
    return out


# speedup vs baseline: 1.0153x; 1.0153x over previous
import functools

import jax
import jax.numpy as jnp
from jax import lax
from jax.experimental import pallas as pl
from jax.experimental.pallas import tpu as pltpu
from jax.experimental.pallas import tpu_sc as plsc

N = 10000
E = 320000
D = 128
HD = D // 2
G = 64

NC = 2
NS = 16
NW = NC * NS

CHR = 200
ROWS = E // CHR
RPT = N // NS

DEG_CPT = ROWS // NW
DEG_NB = 5
DEG_NGRP = DEG_CPT // DEG_NB

EDG_CPT = ROWS // NS
EDG_NPH = 2
EDG_PCH = EDG_CPT // EDG_NPH
EDG_NB = 5
EDG_NGRP = EDG_PCH // EDG_NB
CO_K = 5
CO_R = RPT // CO_K

BN = 1000
NBLK = N // BN

_mesh = plsc.VectorSubcoreMesh(core_axis_name="c", subcore_axis_name="s")
_sc_params = pltpu.CompilerParams(use_tc_tiling_on_sc=False)


@functools.partial(
    pl.kernel,
    out_type=jax.ShapeDtypeStruct((NC, N, 16), jnp.float32),
    mesh=_mesh,
    scratch_types=[
        pltpu.VMEM((DEG_CPT, CHR), jnp.int32),
        pltpu.VMEM((CHR, 16), jnp.float32),
        pltpu.VMEM_SHARED((N, 16), jnp.float32),
        pltpu.SemaphoreType.DMA,
        pltpu.SemaphoreType.DMA,
        pltpu.SemaphoreType.DMA,
        pltpu.SemaphoreType.DMA,
        pltpu.SemaphoreType.DMA,
    ],
    compiler_params=_sc_params,
)
def _sc_degree(dst_hbm, ones_hbm, zeros_hbm, out_hbm,
               idxd_v, ones_v, acc_sh, s0, s1, s2, s3, s4):
    ssem = (s0, s1, s2, s3, s4)
    cid = lax.axis_index("c")
    sid = lax.axis_index("s")
    wid = sid * NC + cid
    rbase = sid * RPT
    pltpu.sync_copy(zeros_hbm.at[pl.ds(rbase, RPT)],
                    acc_sh.at[pl.ds(rbase, RPT)])
    pltpu.sync_copy(ones_hbm, ones_v)
    pltpu.sync_copy(dst_hbm.at[pl.ds(wid * DEG_CPT, DEG_CPT)], idxd_v)
    plsc.subcore_barrier()

    def scat(c, b):
        pltpu.async_copy(ones_v, acc_sh.at[idxd_v.at[c]], ssem[b], add=True)

    def swait(c, b):
        pltpu.make_async_copy(ones_v, acc_sh.at[idxd_v.at[c]], ssem[b]).wait()

    def group(j0, carry):
        j = j0 * DEG_NB
        for b in range(DEG_NB):
            scat(j + b, b)
        for b in range(DEG_NB):
            swait(j + b, b)
        return carry

    lax.fori_loop(0, DEG_NGRP, group, 0)

    plsc.subcore_barrier()
    pltpu.sync_copy(acc_sh.at[pl.ds(rbase, RPT)],
                    out_hbm.at[cid, pl.ds(rbase, RPT)])


@functools.partial(
    pl.kernel,
    out_type=jax.ShapeDtypeStruct((2 * N, HD), jnp.float32),
    mesh=_mesh,
    scratch_types=[
        pltpu.VMEM((EDG_PCH, CHR), jnp.int32),
        pltpu.VMEM((EDG_PCH, CHR), jnp.int32),
        pltpu.VMEM((EDG_NB, CHR, HD), jnp.float32),
        pltpu.VMEM((CO_K, CO_R), jnp.int32),
        pltpu.VMEM_SHARED((N, HD), jnp.float32),
    ] + [pltpu.SemaphoreType.DMA] * (2 * EDG_NB),
    compiler_params=_sc_params,
)
def _sc_edge_sum(y2n_hbm, src2_hbm, dst_hbm, coidx_hbm, zeros_hbm, out_hbm,
                 idxs_v, idxd_v, rows_v, coidx_v, acc_sh,
                 *sems):
    gsem = sems[:EDG_NB]
    ssem = sems[EDG_NB:]
    cid = lax.axis_index("c")
    sid = lax.axis_index("s")
    rbase = sid * RPT
    pltpu.sync_copy(zeros_hbm.at[pl.ds(rbase, RPT)],
                    acc_sh.at[pl.ds(rbase, RPT)])
    pltpu.sync_copy(coidx_hbm.at[cid, sid], coidx_v)
    cbase = sid * EDG_CPT

    def gath(c, b):
        pltpu.async_copy(y2n_hbm.at[idxs_v.at[c]], rows_v.at[b], gsem[b])

    def gwait(c, b):
        pltpu.make_async_copy(y2n_hbm.at[idxs_v.at[c]], rows_v.at[b],
                              gsem[b]).wait()

    def scat(c, b):
        pltpu.async_copy(rows_v.at[b], acc_sh.at[idxd_v.at[c]], ssem[b],
                         add=True)

    def swait(c, b):
        pltpu.make_async_copy(rows_v.at[b], acc_sh.at[idxd_v.at[c]],
                              ssem[b]).wait()

    for p in range(EDG_NPH):
        pbase = cbase + p * EDG_PCH
        pltpu.sync_copy(src2_hbm.at[cid, pl.ds(pbase, EDG_PCH)], idxs_v)
        pltpu.sync_copy(dst_hbm.at[pl.ds(pbase, EDG_PCH)], idxd_v)
        if p == 0:
            plsc.subcore_barrier()

        for b in range(EDG_NB):
            gath(b, b)

        def group(j0, carry):
            j = j0 * EDG_NB
            for b in range(EDG_NB):
                gwait(j + b, b)
                scat(j + b, b)
            for b in range(EDG_NB):
                swait(j + b, b)
                gath(j + EDG_NB + b, b)
            return carry

        lax.fori_loop(0, EDG_NGRP - 1, group, 0)
        j = (EDG_NGRP - 1) * EDG_NB
        for b in range(EDG_NB):
            gwait(j + b, b)
            scat(j + b, b)
        for b in range(EDG_NB):
            swait(j + b, b)

    plsc.subcore_barrier()
    for k in range(CO_K):
        stage = rows_v.at[k].at[pl.ds(0, CO_R)]
        pltpu.sync_copy(acc_sh.at[pl.ds(rbase + k * CO_R, CO_R)], stage)
        pltpu.async_copy(stage, out_hbm.at[coidx_v.at[k]], ssem[0]).wait()


def _dvec(degp_ref):
    deg = 1.0 + degp_ref[0, :, :1] + degp_ref[1, :, :1]
    return lax.rsqrt(deg)


def _tc1a_body(x_ref, w1_ref, z_ref):
    z_ref[...] = jnp.dot(x_ref[...], w1_ref[...],
                         preferred_element_type=jnp.float32,
                         precision=lax.Precision.HIGHEST)


def _tc1b_body(degp_ref, z_ref, y1_ref):
    y1_ref[...] = z_ref[...] * _dvec(degp_ref)


def _tc2_body(degp_ref, s_ref, y1_ref, w2_ref, b1_ref, y2_ref):
    d = _dvec(degp_ref)
    s = s_ref[...] + y1_ref[...]
    h = jnp.maximum(d * s + b1_ref[...], 0.0)
    hw = jnp.dot(h, w2_ref[...],
                 preferred_element_type=jnp.float32,
                 precision=lax.Precision.HIGHEST)
    y2_ref[...] = hw * d


def _tc3_body(degp_ref, s_ref, y2_ref, b2_ref, batch_ref, out_ref, acc, cnt):
    i = pl.program_id(0)

    @pl.when(i == 0)
    def _init():
        acc[...] = jnp.zeros_like(acc)
        cnt[...] = jnp.zeros_like(cnt)

    d = _dvec(degp_ref)
    o = d * (s_ref[...] + y2_ref[...]) + b2_ref[...]
    seg = batch_ref[0, 0, :]
    oh = (lax.broadcasted_iota(jnp.int32, (G, BN), 0)
          == seg[None, :]).astype(jnp.float32)
    acc[...] += jnp.dot(oh, o, preferred_element_type=jnp.float32,
                        precision=lax.Precision.HIGHEST)
    cnt[...] += jnp.sum(oh, axis=1, keepdims=True)

    @pl.when(i == NBLK - 1)
    def _fin():
        out_ref[...] = acc[...] / jnp.maximum(cnt[...], 1.0)


_row = lambda i: (i, 0)
_fix2 = lambda i: (0, 0)
_split3 = lambda i: (0, i, 0)

_tc1a = pl.pallas_call(
    _tc1a_body,
    grid=(NBLK,),
    in_specs=[
        pl.BlockSpec((BN, D), _row),
        pl.BlockSpec((D, D), _fix2),
    ],
    out_specs=pl.BlockSpec((BN, D), _row),
    out_shape=jax.ShapeDtypeStruct((N, D), jnp.float32),
)

_tc1b = pl.pallas_call(
    _tc1b_body,
    grid=(NBLK,),
    in_specs=[
        pl.BlockSpec((NC, BN, 16), _split3),
        pl.BlockSpec((BN, D), _row),
    ],
    out_specs=pl.BlockSpec((BN, D), _row),
    out_shape=jax.ShapeDtypeStruct((N, D), jnp.float32),
)

_tc2 = pl.pallas_call(
    _tc2_body,
    grid=(NBLK,),
    in_specs=[
        pl.BlockSpec((NC, BN, 16), _split3),
        pl.BlockSpec((BN, D), _row),
        pl.BlockSpec((BN, D), _row),
        pl.BlockSpec((D, D), _fix2),
        pl.BlockSpec((1, D), _fix2),
    ],
    out_specs=pl.BlockSpec((BN, D), _row),
    out_shape=jax.ShapeDtypeStruct((N, D), jnp.float32),
)

_tc3 = pl.pallas_call(
    _tc3_body,
    grid=(NBLK,),
    in_specs=[
        pl.BlockSpec((NC, BN, 16), _split3),
        pl.BlockSpec((BN, D), _row),
        pl.BlockSpec((BN, D), _row),
        pl.BlockSpec((1, D), _fix2),
        pl.BlockSpec((1, 1, BN), lambda i: (i, 0, 0)),
    ],
    out_specs=pl.BlockSpec((G, D), _fix2),
    out_shape=jax.ShapeDtypeStruct((G, D), jnp.float32),
    scratch_shapes=[
        pltpu.VMEM((G, D), jnp.float32),
        pltpu.VMEM((G, 1), jnp.float32),
    ],
)


def kernel(x, edge_index, batch, W1, b1, W2, b2):
    x = x.astype(jnp.float32)
    src2d = edge_index[0].reshape(ROWS, CHR)
    dst2d = edge_index[1].reshape(ROWS, CHR)
    src2 = jnp.stack([src2d * 2, src2d * 2 + 1])
    co = 2 * jnp.arange(N, dtype=jnp.int32).reshape(NS, CO_K, CO_R)
    coidx = jnp.stack([co, co + 1])
    ones16 = jnp.ones((CHR, 16), jnp.float32)
    zeros16 = jnp.zeros((N, 16), jnp.float32)
    zerosH = jnp.zeros((N, HD), jnp.float32)
    b1r = b1.reshape(1, D)
    b2r = b2.reshape(1, D)
    batch3 = batch.reshape(NBLK, 1, BN)

    degp = _sc_degree(dst2d, ones16, zeros16)
    z1 = _tc1a(x, W1)
    y1 = _tc1b(degp, z1)
    s1 = _sc_edge_sum(y1.reshape(2 * N, HD), src2, dst2d, coidx,
                      zerosH).reshape(N, D)
    y2 = _tc2(degp, s1, y1, W2, b1r)
    s2 = _sc_edge_sum(y2.reshape(2 * N, HD), src2, dst2d, coidx,
                      zerosH).reshape(N, D)
    out = _tc3(degp, s2, y2, b2r, batch3)
    return out


# speedup vs baseline: 1.0437x; 1.0279x over previous
import functools

import jax
import jax.numpy as jnp
from jax import lax
from jax.experimental import pallas as pl
from jax.experimental.pallas import tpu as pltpu
from jax.experimental.pallas import tpu_sc as plsc

N = 10000
E = 320000
D = 128
HD = D // 2
G = 64

NC = 2
NS = 16
NW = NC * NS

CHR = 200
ROWS = E // CHR
RPT = N // NS

DEG_CPT = ROWS // NW
DEG_NB = 5
DEG_NGRP = DEG_CPT // DEG_NB

EDG_CPT = ROWS // NS
EDG_NPH = 2
EDG_PCH = EDG_CPT // EDG_NPH
EDG_NB = 5
EDG_NGRP = EDG_PCH // EDG_NB
CO_K = 5
CO_R = RPT // CO_K

BN = 1000
NBLK = N // BN

_mesh = plsc.VectorSubcoreMesh(core_axis_name="c", subcore_axis_name="s")
_sc_params = pltpu.CompilerParams(use_tc_tiling_on_sc=False)


@functools.partial(
    pl.kernel,
    out_type=jax.ShapeDtypeStruct((NC, N, 16), jnp.float32),
    mesh=_mesh,
    scratch_types=[
        pltpu.VMEM((DEG_CPT, CHR), jnp.int32),
        pltpu.VMEM((CHR, 16), jnp.float32),
        pltpu.VMEM_SHARED((N, 16), jnp.float32),
        pltpu.SemaphoreType.DMA,
        pltpu.SemaphoreType.DMA,
        pltpu.SemaphoreType.DMA,
        pltpu.SemaphoreType.DMA,
        pltpu.SemaphoreType.DMA,
    ],
    compiler_params=_sc_params,
)
def _sc_degree(dst_hbm, ones_hbm, zeros_hbm, out_hbm,
               idxd_v, ones_v, acc_sh, s0, s1, s2, s3, s4):
    ssem = (s0, s1, s2, s3, s4)
    cid = lax.axis_index("c")
    sid = lax.axis_index("s")
    wid = sid * NC + cid
    rbase = sid * RPT
    pltpu.sync_copy(zeros_hbm.at[pl.ds(rbase, RPT)],
                    acc_sh.at[pl.ds(rbase, RPT)])
    pltpu.sync_copy(ones_hbm, ones_v)
    pltpu.sync_copy(dst_hbm.at[pl.ds(wid * DEG_CPT, DEG_CPT)], idxd_v)
    plsc.subcore_barrier()

    def scat(c, b):
        pltpu.async_copy(ones_v, acc_sh.at[idxd_v.at[c]], ssem[b], add=True)

    def swait(c, b):
        pltpu.make_async_copy(ones_v, acc_sh.at[idxd_v.at[c]], ssem[b]).wait()

    def group(j0, carry):
        j = j0 * DEG_NB
        for b in range(DEG_NB):
            scat(j + b, b)
        for b in range(DEG_NB):
            swait(j + b, b)
        return carry

    lax.fori_loop(0, DEG_NGRP, group, 0)

    plsc.subcore_barrier()
    pltpu.sync_copy(acc_sh.at[pl.ds(rbase, RPT)],
                    out_hbm.at[cid, pl.ds(rbase, RPT)])


@functools.partial(
    pl.kernel,
    out_type=jax.ShapeDtypeStruct((2 * N, HD), jnp.float32),
    mesh=_mesh,
    scratch_types=[
        pltpu.VMEM((EDG_PCH, CHR), jnp.int32),
        pltpu.VMEM((EDG_PCH, CHR), jnp.int32),
        pltpu.VMEM((EDG_NB, CHR, HD), jnp.float32),
        pltpu.VMEM((CO_K, CO_R), jnp.int32),
        pltpu.VMEM_SHARED((N, HD), jnp.float32),
    ] + [pltpu.SemaphoreType.DMA] * (2 * EDG_NB),
    compiler_params=_sc_params,
)
def _sc_edge_sum(y2n_hbm, src2_hbm, dst_hbm, coidx_hbm, zeros_hbm, out_hbm,
                 idxs_v, idxd_v, rows_v, coidx_v, acc_sh,
                 *sems):
    gsem = sems[:EDG_NB]
    ssem = sems[EDG_NB:]
    cid = lax.axis_index("c")
    sid = lax.axis_index("s")
    rbase = sid * RPT
    pltpu.sync_copy(zeros_hbm.at[pl.ds(rbase, RPT)],
                    acc_sh.at[pl.ds(rbase, RPT)])
    pltpu.sync_copy(coidx_hbm.at[cid, sid], coidx_v)
    cbase = sid * EDG_CPT

    def gath(c, b):
        pltpu.async_copy(y2n_hbm.at[idxs_v.at[c]], rows_v.at[b], gsem[b])

    def gwait(c, b):
        pltpu.make_async_copy(y2n_hbm.at[idxs_v.at[c]], rows_v.at[b],
                              gsem[b]).wait()

    def scat(c, b):
        pltpu.async_copy(rows_v.at[b], acc_sh.at[idxd_v.at[c]], ssem[b],
                         add=True)

    def swait(c, b):
        pltpu.make_async_copy(rows_v.at[b], acc_sh.at[idxd_v.at[c]],
                              ssem[b]).wait()

    for p in range(EDG_NPH):
        pbase = cbase + p * EDG_PCH
        pltpu.sync_copy(src2_hbm.at[cid, pl.ds(pbase, EDG_PCH)], idxs_v)
        pltpu.sync_copy(dst_hbm.at[pl.ds(pbase, EDG_PCH)], idxd_v)
        if p == 0:
            plsc.subcore_barrier()

        for b in range(EDG_NB):
            gath(b, b)

        def group(j0, carry):
            j = j0 * EDG_NB
            for b in range(EDG_NB):
                gwait(j + b, b)
                scat(j + b, b)
            for b in range(EDG_NB):
                swait(j + b, b)
                gath(j + EDG_NB + b, b)
            return carry

        lax.fori_loop(0, EDG_NGRP - 1, group, 0)
        j = (EDG_NGRP - 1) * EDG_NB
        for b in range(EDG_NB):
            gwait(j + b, b)
            scat(j + b, b)
        for b in range(EDG_NB):
            swait(j + b, b)

    plsc.subcore_barrier()
    for k in range(CO_K):
        pltpu.async_copy(acc_sh.at[pl.ds(rbase + k * CO_R, CO_R)],
                         rows_v.at[k].at[pl.ds(0, CO_R)], gsem[k])
    for k in range(CO_K):
        stage = rows_v.at[k].at[pl.ds(0, CO_R)]
        pltpu.make_async_copy(acc_sh.at[pl.ds(rbase + k * CO_R, CO_R)],
                              stage, gsem[k]).wait()
        pltpu.async_copy(stage, out_hbm.at[coidx_v.at[k]], ssem[k])
    for k in range(CO_K):
        pltpu.make_async_copy(rows_v.at[k].at[pl.ds(0, CO_R)],
                              out_hbm.at[coidx_v.at[k]], ssem[k]).wait()


def _dvec(degp_ref):
    deg = 1.0 + degp_ref[0, :, :1] + degp_ref[1, :, :1]
    return lax.rsqrt(deg)


def _tc1_body(degp_ref, x_ref, w1_ref, y1_ref):
    d = _dvec(degp_ref)
    xw = jnp.dot(x_ref[...], w1_ref[...],
                 preferred_element_type=jnp.float32,
                 precision=lax.Precision.HIGHEST)
    y1_ref[...] = xw * d


def _tc2_body(degp_ref, s_ref, y1_ref, w2_ref, b1_ref, y2_ref):
    d = _dvec(degp_ref)
    s = s_ref[...] + y1_ref[...]
    h = jnp.maximum(d * s + b1_ref[...], 0.0)
    hw = jnp.dot(h, w2_ref[...],
                 preferred_element_type=jnp.float32,
                 precision=lax.Precision.HIGHEST)
    y2_ref[...] = hw * d


def _tc3_body(degp_ref, s_ref, y2_ref, b2_ref, batch_ref, out_ref, acc, cnt):
    i = pl.program_id(0)

    @pl.when(i == 0)
    def _init():
        acc[...] = jnp.zeros_like(acc)
        cnt[...] = jnp.zeros_like(cnt)

    d = _dvec(degp_ref)
    o = d * (s_ref[...] + y2_ref[...]) + b2_ref[...]
    seg = batch_ref[0, 0, :]
    oh = (lax.broadcasted_iota(jnp.int32, (G, BN), 0)
          == seg[None, :]).astype(jnp.float32)
    acc[...] += jnp.dot(oh, o, preferred_element_type=jnp.float32,
                        precision=lax.Precision.HIGHEST)
    cnt[...] += jnp.sum(oh, axis=1, keepdims=True)

    @pl.when(i == NBLK - 1)
    def _fin():
        out_ref[...] = acc[...] / jnp.maximum(cnt[...], 1.0)


_row = lambda i: (i, 0)
_fix2 = lambda i: (0, 0)
_split3 = lambda i: (0, i, 0)

_tc1 = pl.pallas_call(
    _tc1_body,
    grid=(NBLK,),
    in_specs=[
        pl.BlockSpec((NC, BN, 16), _split3),
        pl.BlockSpec((BN, D), _row),
        pl.BlockSpec((D, D), _fix2),
    ],
    out_specs=pl.BlockSpec((BN, D), _row),
    out_shape=jax.ShapeDtypeStruct((N, D), jnp.float32),
)

_tc2 = pl.pallas_call(
    _tc2_body,
    grid=(NBLK,),
    in_specs=[
        pl.BlockSpec((NC, BN, 16), _split3),
        pl.BlockSpec((BN, D), _row),
        pl.BlockSpec((BN, D), _row),
        pl.BlockSpec((D, D), _fix2),
        pl.BlockSpec((1, D), _fix2),
    ],
    out_specs=pl.BlockSpec((BN, D), _row),
    out_shape=jax.ShapeDtypeStruct((N, D), jnp.float32),
)

_tc3 = pl.pallas_call(
    _tc3_body,
    grid=(NBLK,),
    in_specs=[
        pl.BlockSpec((NC, BN, 16), _split3),
        pl.BlockSpec((BN, D), _row),
        pl.BlockSpec((BN, D), _row),
        pl.BlockSpec((1, D), _fix2),
        pl.BlockSpec((1, 1, BN), lambda i: (i, 0, 0)),
    ],
    out_specs=pl.BlockSpec((G, D), _fix2),
    out_shape=jax.ShapeDtypeStruct((G, D), jnp.float32),
    scratch_shapes=[
        pltpu.VMEM((G, D), jnp.float32),
        pltpu.VMEM((G, 1), jnp.float32),
    ],
)


def kernel(x, edge_index, batch, W1, b1, W2, b2):
    x = x.astype(jnp.float32)
    src2d = edge_index[0].reshape(ROWS, CHR)
    dst2d = edge_index[1].reshape(ROWS, CHR)
    src2 = jnp.stack([src2d * 2, src2d * 2 + 1])
    co = 2 * jnp.arange(N, dtype=jnp.int32).reshape(NS, CO_K, CO_R)
    coidx = jnp.stack([co, co + 1])
    ones16 = jnp.ones((CHR, 16), jnp.float32)
    zeros16 = jnp.zeros((N, 16), jnp.float32)
    zerosH = jnp.zeros((N, HD), jnp.float32)
    b1r = b1.reshape(1, D)
    b2r = b2.reshape(1, D)
    batch3 = batch.reshape(NBLK, 1, BN)

    degp = _sc_degree(dst2d, ones16, zeros16)
    y1 = _tc1(degp, x, W1)
    s1 = _sc_edge_sum(y1.reshape(2 * N, HD), src2, dst2d, coidx,
                      zerosH).reshape(N, D)
    y2 = _tc2(degp, s1, y1, W2, b1r)
    s2 = _sc_edge_sum(y2.reshape(2 * N, HD), src2, dst2d, coidx,
                      zerosH).reshape(N, D)
    out = _tc3(degp, s2, y2, b2r, batch3)
    return out


# speedup vs baseline: 1.0566x; 1.0124x over previous
import functools

import jax
import jax.numpy as jnp
from jax import lax
from jax.experimental import pallas as pl
from jax.experimental.pallas import tpu as pltpu
from jax.experimental.pallas import tpu_sc as plsc

N = 10000
E = 320000
D = 128
HD = D // 2
G = 64

NC = 2
NS = 16
NW = NC * NS

CHR = 200
ROWS = E // CHR
RPT = N // NS

DEG_CPT = ROWS // NW
DEG_NB = 5
DEG_NGRP = DEG_CPT // DEG_NB

EDG_CPT = ROWS // NS
EDG_NPH = 2
EDG_PCH = EDG_CPT // EDG_NPH
EDG_NB = 5
EDG_NGRP = EDG_PCH // EDG_NB
CO_K = 5
CO_R = RPT // CO_K

BN = 1000
NBLK = N // BN

_mesh = plsc.VectorSubcoreMesh(core_axis_name="c", subcore_axis_name="s")
_sc_params = pltpu.CompilerParams(use_tc_tiling_on_sc=False)


@functools.partial(
    pl.kernel,
    out_type=jax.ShapeDtypeStruct((NC, N, 16), jnp.float32),
    mesh=_mesh,
    scratch_types=[
        pltpu.VMEM((DEG_CPT, CHR), jnp.int32),
        pltpu.VMEM((CHR, 16), jnp.float32),
        pltpu.VMEM_SHARED((N, 16), jnp.float32),
        pltpu.SemaphoreType.DMA,
        pltpu.SemaphoreType.DMA,
        pltpu.SemaphoreType.DMA,
        pltpu.SemaphoreType.DMA,
        pltpu.SemaphoreType.DMA,
    ],
    compiler_params=_sc_params,
)
def _sc_degree(dst_hbm, ones_hbm, zeros_hbm, out_hbm,
               idxd_v, ones_v, acc_sh, s0, s1, s2, s3, s4):
    ssem = (s0, s1, s2, s3, s4)
    cid = lax.axis_index("c")
    sid = lax.axis_index("s")
    wid = sid * NC + cid
    rbase = sid * RPT
    pltpu.sync_copy(zeros_hbm.at[pl.ds(rbase, RPT)],
                    acc_sh.at[pl.ds(rbase, RPT)])
    pltpu.sync_copy(ones_hbm, ones_v)
    pltpu.sync_copy(dst_hbm.at[pl.ds(wid * DEG_CPT, DEG_CPT)], idxd_v)
    plsc.subcore_barrier()

    def scat(c, b):
        pltpu.async_copy(ones_v, acc_sh.at[idxd_v.at[c]], ssem[b], add=True)

    def swait(c, b):
        pltpu.make_async_copy(ones_v, acc_sh.at[idxd_v.at[c]], ssem[b]).wait()

    def group(j0, carry):
        j = j0 * DEG_NB
        for b in range(DEG_NB):
            scat(j + b, b)
        for b in range(DEG_NB):
            swait(j + b, b)
        return carry

    lax.fori_loop(0, DEG_NGRP, group, 0)

    plsc.subcore_barrier()
    pltpu.sync_copy(acc_sh.at[pl.ds(rbase, RPT)],
                    out_hbm.at[cid, pl.ds(rbase, RPT)])


@functools.partial(
    pl.kernel,
    out_type=jax.ShapeDtypeStruct((2 * N, HD), jnp.float32),
    mesh=_mesh,
    scratch_types=[
        pltpu.VMEM((EDG_PCH, CHR), jnp.int32),
        pltpu.VMEM((EDG_PCH, CHR), jnp.int32),
        pltpu.VMEM((EDG_NB, CHR, HD), jnp.float32),
        pltpu.VMEM((CO_K, CO_R), jnp.int32),
        pltpu.VMEM_SHARED((N, HD), jnp.float32),
    ] + [pltpu.SemaphoreType.DMA] * (2 * EDG_NB),
    compiler_params=_sc_params,
)
def _sc_edge_sum(y2n_hbm, src2_hbm, dst_hbm, coidx_hbm, zeros_hbm, out_hbm,
                 idxs_v, idxd_v, rows_v, coidx_v, acc_sh,
                 *sems):
    gsem = sems[:EDG_NB]
    ssem = sems[EDG_NB:]
    cid = lax.axis_index("c")
    sid = lax.axis_index("s")
    rbase = sid * RPT
    pltpu.async_copy(zeros_hbm.at[pl.ds(rbase, RPT)],
                     acc_sh.at[pl.ds(rbase, RPT)], sems[EDG_NB])
    pltpu.sync_copy(coidx_hbm.at[cid, sid], coidx_v)
    cbase = sid * EDG_CPT

    def gath(c, b):
        pltpu.async_copy(y2n_hbm.at[idxs_v.at[c]], rows_v.at[b], gsem[b])

    def gwait(c, b):
        pltpu.make_async_copy(y2n_hbm.at[idxs_v.at[c]], rows_v.at[b],
                              gsem[b]).wait()

    def scat(c, b):
        pltpu.async_copy(rows_v.at[b], acc_sh.at[idxd_v.at[c]], ssem[b],
                         add=True)

    def swait(c, b):
        pltpu.make_async_copy(rows_v.at[b], acc_sh.at[idxd_v.at[c]],
                              ssem[b]).wait()

    for p in range(EDG_NPH):
        pbase = cbase + p * EDG_PCH
        pltpu.sync_copy(src2_hbm.at[cid, pl.ds(pbase, EDG_PCH)], idxs_v)
        pltpu.sync_copy(dst_hbm.at[pl.ds(pbase, EDG_PCH)], idxd_v)

        for b in range(EDG_NB):
            gath(b, b)

        if p == 0:
            pltpu.make_async_copy(zeros_hbm.at[pl.ds(rbase, RPT)],
                                  acc_sh.at[pl.ds(rbase, RPT)],
                                  sems[EDG_NB]).wait()
            plsc.subcore_barrier()

        def group(j0, carry):
            j = j0 * EDG_NB
            for b in range(EDG_NB):
                gwait(j + b, b)
                scat(j + b, b)
            for b in range(EDG_NB):
                swait(j + b, b)
                gath(j + EDG_NB + b, b)
            return carry

        lax.fori_loop(0, EDG_NGRP - 1, group, 0)
        j = (EDG_NGRP - 1) * EDG_NB
        for b in range(EDG_NB):
            gwait(j + b, b)
            scat(j + b, b)
        for b in range(EDG_NB):
            swait(j + b, b)

    plsc.subcore_barrier()
    for k in range(CO_K):
        pltpu.async_copy(acc_sh.at[pl.ds(rbase + k * CO_R, CO_R)],
                         rows_v.at[k].at[pl.ds(0, CO_R)], gsem[k])
    for k in range(CO_K):
        stage = rows_v.at[k].at[pl.ds(0, CO_R)]
        pltpu.make_async_copy(acc_sh.at[pl.ds(rbase + k * CO_R, CO_R)],
                              stage, gsem[k]).wait()
        pltpu.async_copy(stage, out_hbm.at[coidx_v.at[k]], ssem[k])
    for k in range(CO_K):
        pltpu.make_async_copy(rows_v.at[k].at[pl.ds(0, CO_R)],
                              out_hbm.at[coidx_v.at[k]], ssem[k]).wait()


def _dvec(degp_ref):
    deg = 1.0 + degp_ref[0, :, :1] + degp_ref[1, :, :1]
    return lax.rsqrt(deg)


def _tc1_body(degp_ref, x_ref, w1_ref, y1_ref):
    d = _dvec(degp_ref)
    xw = jnp.dot(x_ref[...], w1_ref[...],
                 preferred_element_type=jnp.float32,
                 precision=lax.Precision.HIGHEST)
    y1_ref[...] = xw * d


def _tc2_body(degp_ref, s_ref, y1_ref, w2_ref, b1_ref, y2_ref):
    d = _dvec(degp_ref)
    s = s_ref[...] + y1_ref[...]
    h = jnp.maximum(d * s + b1_ref[...], 0.0)
    hw = jnp.dot(h, w2_ref[...],
                 preferred_element_type=jnp.float32,
                 precision=lax.Precision.HIGHEST)
    y2_ref[...] = hw * d


def _tc3_body(degp_ref, s_ref, y2_ref, b2_ref, batch_ref, out_ref, acc, cnt):
    i = pl.program_id(0)

    @pl.when(i == 0)
    def _init():
        acc[...] = jnp.zeros_like(acc)
        cnt[...] = jnp.zeros_like(cnt)

    d = _dvec(degp_ref)
    o = d * (s_ref[...] + y2_ref[...]) + b2_ref[...]
    seg = batch_ref[0, 0, :]
    oh = (lax.broadcasted_iota(jnp.int32, (G, BN), 0)
          == seg[None, :]).astype(jnp.float32)
    acc[...] += jnp.dot(oh, o, preferred_element_type=jnp.float32,
                        precision=lax.Precision.HIGHEST)
    cnt[...] += jnp.sum(oh, axis=1, keepdims=True)

    @pl.when(i == NBLK - 1)
    def _fin():
        out_ref[...] = acc[...] / jnp.maximum(cnt[...], 1.0)


_row = lambda i: (i, 0)
_fix2 = lambda i: (0, 0)
_split3 = lambda i: (0, i, 0)

_tc1 = pl.pallas_call(
    _tc1_body,
    grid=(NBLK,),
    in_specs=[
        pl.BlockSpec((NC, BN, 16), _split3),
        pl.BlockSpec((BN, D), _row),
        pl.BlockSpec((D, D), _fix2),
    ],
    out_specs=pl.BlockSpec((BN, D), _row),
    out_shape=jax.ShapeDtypeStruct((N, D), jnp.float32),
)

_tc2 = pl.pallas_call(
    _tc2_body,
    grid=(NBLK,),
    in_specs=[
        pl.BlockSpec((NC, BN, 16), _split3),
        pl.BlockSpec((BN, D), _row),
        pl.BlockSpec((BN, D), _row),
        pl.BlockSpec((D, D), _fix2),
        pl.BlockSpec((1, D), _fix2),
    ],
    out_specs=pl.BlockSpec((BN, D), _row),
    out_shape=jax.ShapeDtypeStruct((N, D), jnp.float32),
)

_tc3 = pl.pallas_call(
    _tc3_body,
    grid=(NBLK,),
    in_specs=[
        pl.BlockSpec((NC, BN, 16), _split3),
        pl.BlockSpec((BN, D), _row),
        pl.BlockSpec((BN, D), _row),
        pl.BlockSpec((1, D), _fix2),
        pl.BlockSpec((1, 1, BN), lambda i: (i, 0, 0)),
    ],
    out_specs=pl.BlockSpec((G, D), _fix2),
    out_shape=jax.ShapeDtypeStruct((G, D), jnp.float32),
    scratch_shapes=[
        pltpu.VMEM((G, D), jnp.float32),
        pltpu.VMEM((G, 1), jnp.float32),
    ],
)


def kernel(x, edge_index, batch, W1, b1, W2, b2):
    x = x.astype(jnp.float32)
    src2d = edge_index[0].reshape(ROWS, CHR)
    dst2d = edge_index[1].reshape(ROWS, CHR)
    src2 = jnp.stack([src2d * 2, src2d * 2 + 1])
    co = 2 * jnp.arange(N, dtype=jnp.int32).reshape(NS, CO_K, CO_R)
    coidx = jnp.stack([co, co + 1])
    ones16 = jnp.ones((CHR, 16), jnp.float32)
    zeros16 = jnp.zeros((N, 16), jnp.float32)
    zerosH = jnp.zeros((N, HD), jnp.float32)
    b1r = b1.reshape(1, D)
    b2r = b2.reshape(1, D)
    batch3 = batch.reshape(NBLK, 1, BN)

    degp = _sc_degree(dst2d, ones16, zeros16)
    y1 = _tc1(degp, x, W1)
    s1 = _sc_edge_sum(y1.reshape(2 * N, HD), src2, dst2d, coidx,
                      zerosH).reshape(N, D)
    y2 = _tc2(degp, s1, y1, W2, b1r)
    s2 = _sc_edge_sum(y2.reshape(2 * N, HD), src2, dst2d, coidx,
                      zerosH).reshape(N, D)
    out = _tc3(degp, s2, y2, b2r, batch3)
    return out
